# Initial kernel scaffold; baseline (speedup 1.0000x reference)
import jax
import jax.numpy as jnp
from jax.experimental import pallas as pl


def kernel(x, params):
    raise NotImplementedError("write your pallas kernel here")


# speedup vs baseline: 3.3460x; 3.3460x over previous
import functools

import jax
import jax.numpy as jnp
from jax import lax
from jax.experimental import pallas as pl
from jax.experimental.pallas import tpu as pltpu

K_NB = 16
EPS = 1e-5
B, N, C = 8, 1024, 64
BLK = 256
CCHUNK = 4
F32 = jnp.float32


def _agg_coef():
    dsib = (F32(1.0) / jnp.sqrt(F32(17.0))).astype(jnp.bfloat16).astype(F32)
    return dsib * dsib


def _gcn_layer_kernel(feats_ref, fblk_ref, w_ref, b_ref, g_ref, be_ref,
                      out_ref, outall_ref, ft_ref, fbt_ref, d_ref, m_ref,
                      *, dil, n_ch):
    nb = pl.program_id(1)
    n_pos = 1 + 15 * dil + 1

    @pl.when(nb == 0)
    def _():
        f = feats_ref[0]
        outall_ref[...] = (
            jnp.dot(f, w_ref[...], preferred_element_type=F32) + b_ref[...])
        ft_ref[...] = jnp.transpose(f)

    fb = fblk_ref[0]
    fbt_ref[...] = jnp.transpose(fb)

    d_ref[...] = jnp.zeros((BLK, N), F32)

    def ch(cc, carry):
        base = cc * CCHUNK
        acc = d_ref[...]
        for k in range(CCHUNK):
            row = ft_ref[pl.ds(base + k, 1), :]
            col = jnp.transpose(fbt_ref[pl.ds(base + k, 1), :])
            dd = col - row
            acc = acc + dd * dd
        d_ref[...] = acc
        return carry

    lax.fori_loop(0, n_ch // CCHUNK, ch, 0)
    d_ref[...] = jnp.sqrt(jnp.maximum(d_ref[...], 0.0))

    iota = lax.broadcasted_iota(jnp.int32, (BLK, N), 1)
    rows = lax.broadcasted_iota(jnp.int32, (BLK, N), 0) + nb * BLK
    coef = _agg_coef()
    m_ref[...] = jnp.where(iota == rows, coef, F32(0.0))
    inf = F32(jnp.inf)

    def body(t, carry):
        dcur = d_ref[...]
        v = jnp.min(dcur, axis=1, keepdims=True)
        isel = jnp.min(jnp.where(dcur == v, iota, N), axis=1, keepdims=True)
        hit = iota == isel
        d_ref[...] = jnp.where(hit, inf, dcur)
        is_s = (t >= 1) & ((t - 1) % dil == 0)

        @pl.when(is_s)
        def _():
            m_ref[...] = m_ref[...] + jnp.where(hit, coef, F32(0.0))
        return carry

    lax.fori_loop(0, n_pos, body, 0)

    msg = jnp.dot(m_ref[...], outall_ref[...], preferred_element_type=F32)
    o = msg / jnp.sqrt(F32(1.0 + EPS)) * g_ref[...] + be_ref[...]
    out_ref[0] = jnp.maximum(o, 0.0)


def _gcn_layer(feats, w, b, g, be, dil, n_ch):
    grid = (B, N // BLK)
    return pl.pallas_call(
        functools.partial(_gcn_layer_kernel, dil=dil, n_ch=n_ch),
        grid=grid,
        in_specs=[
            pl.BlockSpec((1, N, C), lambda bi, nb: (bi, 0, 0)),
            pl.BlockSpec((1, BLK, C), lambda bi, nb: (bi, nb, 0)),
            pl.BlockSpec((C, C), lambda bi, nb: (0, 0)),
            pl.BlockSpec((1, C), lambda bi, nb: (0, 0)),
            pl.BlockSpec((1, C), lambda bi, nb: (0, 0)),
            pl.BlockSpec((1, C), lambda bi, nb: (0, 0)),
        ],
        out_specs=pl.BlockSpec((1, BLK, C), lambda bi, nb: (bi, nb, 0)),
        out_shape=jax.ShapeDtypeStruct((B, N, C), F32),
        scratch_shapes=[
            pltpu.VMEM((N, C), F32),
            pltpu.VMEM((C, N), F32),
            pltpu.VMEM((C, BLK), F32),
            pltpu.VMEM((BLK, N), F32),
            pltpu.VMEM((BLK, N), F32),
        ],
    )(feats, feats, w, b, g, be)


def _head_kernel(f1_ref, f2_ref, f3_ref,
                 w1, b1, g1, be1, w2, b2, g2, be2, w3, b3, g3, be3,
                 w4, b4, g4, be4, w5, b5, g5, be5, w6, b6,
                 out_ref, pooled_ref):
    bi = pl.program_id(0)
    sqc = jnp.sqrt(F32(1.0 + EPS))
    f = (f1_ref[0] + f2_ref[0] + f3_ref[0]) / 3.0
    h = jnp.maximum((jnp.dot(f, w1[...], preferred_element_type=F32)
                     + b1[...]) / sqc * g1[...] + be1[...], 0.0)
    h = jnp.maximum((jnp.dot(h, w2[...], preferred_element_type=F32)
                     + b2[...]) / sqc * g2[...] + be2[...], 0.0)
    h = jnp.maximum((jnp.dot(h, w3[...], preferred_element_type=F32)
                     + b3[...]) / sqc * g3[...] + be3[...], 0.0)
    pooled_ref[pl.ds(bi, 1), :] = jnp.max(h, axis=0, keepdims=True)

    @pl.when(bi == B - 1)
    def _():
        pool = pooled_ref[...]
        c = jnp.maximum((jnp.dot(pool, w4[...], preferred_element_type=F32)
                         + b4[...]) / sqc * g4[...] + be4[...], 0.0)
        c = jnp.maximum((jnp.dot(c, w5[...], preferred_element_type=F32)
                         + b5[...]) / sqc * g5[...] + be5[...], 0.0)
        out_ref[...] = jnp.dot(c, w6[...], preferred_element_type=F32) + b6[...]


def _head(f1, f2, f3, p):
    r2 = lambda a: a.reshape(1, -1)
    args = [f1, f2, f3]
    specs = [pl.BlockSpec((1, N, C), lambda bi: (bi, 0, 0))] * 3
    for j in range(1, 7):
        w = p['l%d_W' % j]
        args.append(w)
        specs.append(pl.BlockSpec(w.shape, lambda bi: (0, 0)))
        bias = r2(p['l%d_b' % j])
        args.append(bias)
        specs.append(pl.BlockSpec(bias.shape, lambda bi: (0, 0)))
        if j < 6:
            for key in ('bn%d_g' % j, 'bn%d_b' % j):
                a = r2(p[key])
                args.append(a)
                specs.append(pl.BlockSpec(a.shape, lambda bi: (0, 0)))
    return pl.pallas_call(
        _head_kernel,
        grid=(B,),
        in_specs=specs,
        out_specs=pl.BlockSpec((B, 40), lambda bi: (0, 0)),
        out_shape=jax.ShapeDtypeStruct((B, 40), F32),
        scratch_shapes=[pltpu.VMEM((B, 1024), F32)],
    )(*args)


def kernel(x, params):
    p = params
    xp = jnp.pad(x, ((0, 0), (0, 0), (0, C - x.shape[-1])))
    w0 = jnp.pad(p['gcn0_W'], ((0, C - p['gcn0_W'].shape[0]), (0, 0)))
    r2 = lambda a: a.reshape(1, -1)
    f1 = _gcn_layer(xp, w0, r2(p['gcn0_b']), r2(p['gcn0_g']),
                    r2(p['gcn0_be']), 1, 4)
    f2 = _gcn_layer(f1, p['gcn1_W'], r2(p['gcn1_b']), r2(p['gcn1_g']),
                    r2(p['gcn1_be']), 2, C)
    f3 = _gcn_layer(f2, p['gcn2_W'], r2(p['gcn2_b']), r2(p['gcn2_g']),
                    r2(p['gcn2_be']), 4, C)
    return _head(f1, f2, f3, p)


# speedup vs baseline: 3.3971x; 1.0153x over previous
import functools

import jax
import jax.numpy as jnp
from jax import lax
from jax.experimental import pallas as pl
from jax.experimental.pallas import tpu as pltpu

K_NB = 16
EPS = 1e-5
B, N, C = 8, 1024, 64
BLK = 256
CCHUNK = 8
F32 = jnp.float32


def _agg_coef():
    dsib = (F32(1.0) / jnp.sqrt(F32(17.0))).astype(jnp.bfloat16).astype(F32)
    return dsib * dsib


def _gcn_layer_kernel(feats_ref, fblk_ref, w_ref, b_ref, g_ref, be_ref,
                      out_ref, outall_ref, ft_ref, fbt_ref, d_ref, m_ref,
                      *, dil, n_ch):
    nb = pl.program_id(1)
    n_pos = 1 + 15 * dil + 1

    @pl.when(nb == 0)
    def _():
        f = feats_ref[0]
        outall_ref[...] = (
            jnp.dot(f, w_ref[...], preferred_element_type=F32) + b_ref[...])
        ft_ref[...] = jnp.transpose(f)

    fb = fblk_ref[0]
    fbt_ref[...] = jnp.transpose(fb)

    d_ref[...] = jnp.zeros((BLK, N), F32)

    def ch(cc, carry):
        base = cc * CCHUNK
        acc = d_ref[...]
        for k in range(CCHUNK):
            row = ft_ref[pl.ds(base + k, 1), :]
            col = jnp.transpose(fbt_ref[pl.ds(base + k, 1), :])
            dd = col - row
            acc = acc + dd * dd
        d_ref[...] = acc
        return carry

    lax.fori_loop(0, n_ch // CCHUNK, ch, 0)
    d_ref[...] = jnp.sqrt(jnp.maximum(d_ref[...], 0.0))

    iota = lax.broadcasted_iota(jnp.int32, (BLK, N), 1)
    rows = lax.broadcasted_iota(jnp.int32, (BLK, N), 0) + nb * BLK
    coef = _agg_coef()
    m_ref[...] = jnp.where(iota == rows, coef, F32(0.0))
    inf = F32(jnp.inf)

    def body(t, carry):
        dcur = d_ref[...]
        v = jnp.min(dcur, axis=1, keepdims=True)
        isel = jnp.min(jnp.where(dcur == v, iota, N), axis=1, keepdims=True)
        hit = iota == isel
        d_ref[...] = jnp.where(hit, inf, dcur)
        is_s = (t >= 1) & ((t - 1) % dil == 0)

        @pl.when(is_s)
        def _():
            m_ref[...] = m_ref[...] + jnp.where(hit, coef, F32(0.0))
        return carry

    lax.fori_loop(0, n_pos, body, 0)

    msg = jnp.dot(m_ref[...], outall_ref[...], preferred_element_type=F32)
    o = msg / jnp.sqrt(F32(1.0 + EPS)) * g_ref[...] + be_ref[...]
    out_ref[0] = jnp.maximum(o, 0.0)


def _gcn_layer(feats, w, b, g, be, dil, n_ch):
    grid = (B, N // BLK)
    return pl.pallas_call(
        functools.partial(_gcn_layer_kernel, dil=dil, n_ch=n_ch),
        grid=grid,
        in_specs=[
            pl.BlockSpec((1, N, C), lambda bi, nb: (bi, 0, 0)),
            pl.BlockSpec((1, BLK, C), lambda bi, nb: (bi, nb, 0)),
            pl.BlockSpec((C, C), lambda bi, nb: (0, 0)),
            pl.BlockSpec((1, C), lambda bi, nb: (0, 0)),
            pl.BlockSpec((1, C), lambda bi, nb: (0, 0)),
            pl.BlockSpec((1, C), lambda bi, nb: (0, 0)),
        ],
        out_specs=pl.BlockSpec((1, BLK, C), lambda bi, nb: (bi, nb, 0)),
        out_shape=jax.ShapeDtypeStruct((B, N, C), F32),
        scratch_shapes=[
            pltpu.VMEM((N, C), F32),
            pltpu.VMEM((C, N), F32),
            pltpu.VMEM((C, BLK), F32),
            pltpu.VMEM((BLK, N), F32),
            pltpu.VMEM((BLK, N), F32),
        ],
    )(feats, feats, w, b, g, be)


def _head_kernel(f1_ref, f2_ref, f3_ref,
                 w1, b1, g1, be1, w2, b2, g2, be2, w3, b3, g3, be3,
                 w4, b4, g4, be4, w5, b5, g5, be5, w6, b6,
                 out_ref, pooled_ref):
    bi = pl.program_id(0)
    sqc = jnp.sqrt(F32(1.0 + EPS))
    f = (f1_ref[0] + f2_ref[0] + f3_ref[0]) / 3.0
    h = jnp.maximum((jnp.dot(f, w1[...], preferred_element_type=F32)
                     + b1[...]) / sqc * g1[...] + be1[...], 0.0)
    h = jnp.maximum((jnp.dot(h, w2[...], preferred_element_type=F32)
                     + b2[...]) / sqc * g2[...] + be2[...], 0.0)
    h = jnp.maximum((jnp.dot(h, w3[...], preferred_element_type=F32)
                     + b3[...]) / sqc * g3[...] + be3[...], 0.0)
    pooled_ref[pl.ds(bi, 1), :] = jnp.max(h, axis=0, keepdims=True)

    @pl.when(bi == B - 1)
    def _():
        pool = pooled_ref[...]
        c = jnp.maximum((jnp.dot(pool, w4[...], preferred_element_type=F32)
                         + b4[...]) / sqc * g4[...] + be4[...], 0.0)
        c = jnp.maximum((jnp.dot(c, w5[...], preferred_element_type=F32)
                         + b5[...]) / sqc * g5[...] + be5[...], 0.0)
        out_ref[...] = jnp.dot(c, w6[...], preferred_element_type=F32) + b6[...]


def _head(f1, f2, f3, p):
    r2 = lambda a: a.reshape(1, -1)
    args = [f1, f2, f3]
    specs = [pl.BlockSpec((1, N, C), lambda bi: (bi, 0, 0))] * 3
    for j in range(1, 7):
        w = p['l%d_W' % j]
        args.append(w)
        specs.append(pl.BlockSpec(w.shape, lambda bi: (0, 0)))
        bias = r2(p['l%d_b' % j])
        args.append(bias)
        specs.append(pl.BlockSpec(bias.shape, lambda bi: (0, 0)))
        if j < 6:
            for key in ('bn%d_g' % j, 'bn%d_b' % j):
                a = r2(p[key])
                args.append(a)
                specs.append(pl.BlockSpec(a.shape, lambda bi: (0, 0)))
    return pl.pallas_call(
        _head_kernel,
        grid=(B,),
        in_specs=specs,
        out_specs=pl.BlockSpec((B, 40), lambda bi: (0, 0)),
        out_shape=jax.ShapeDtypeStruct((B, 40), F32),
        scratch_shapes=[pltpu.VMEM((B, 1024), F32)],
    )(*args)


def kernel(x, params):
    p = params
    xp = jnp.pad(x, ((0, 0), (0, 0), (0, C - x.shape[-1])))
    w0 = jnp.pad(p['gcn0_W'], ((0, C - p['gcn0_W'].shape[0]), (0, 0)))
    r2 = lambda a: a.reshape(1, -1)
    f1 = _gcn_layer(xp, w0, r2(p['gcn0_b']), r2(p['gcn0_g']),
                    r2(p['gcn0_be']), 1, CCHUNK)
    f2 = _gcn_layer(f1, p['gcn1_W'], r2(p['gcn1_b']), r2(p['gcn1_g']),
                    r2(p['gcn1_be']), 2, C)
    f3 = _gcn_layer(f2, p['gcn2_W'], r2(p['gcn2_b']), r2(p['gcn2_g']),
                    r2(p['gcn2_be']), 4, C)
    return _head(f1, f2, f3, p)


# speedup vs baseline: 3.4817x; 1.0249x over previous
import functools

import jax
import jax.numpy as jnp
from jax import lax
from jax.experimental import pallas as pl
from jax.experimental.pallas import tpu as pltpu

K_NB = 16
EPS = 1e-5
B, N, C = 8, 1024, 64
BLK = 256
CCHUNK = 8
F32 = jnp.float32


def _agg_coef():
    dsib = (F32(1.0) / jnp.sqrt(F32(17.0))).astype(jnp.bfloat16).astype(F32)
    return dsib * dsib


def _gcn_layer_kernel(feats_ref, fblk_ref, w_ref, b_ref, g_ref, be_ref,
                      out_ref, outall_ref, ft_ref, fbt_ref, d_ref, m_ref,
                      *, dil, n_ch):
    nb = pl.program_id(1)
    n_pos = 1 + 15 * dil + 1

    @pl.when(nb == 0)
    def _():
        f = feats_ref[0]
        outall_ref[...] = (
            jnp.dot(f, w_ref[...], preferred_element_type=F32) + b_ref[...])
        ft_ref[...] = jnp.transpose(f)

    fb = fblk_ref[0]
    fbt_ref[...] = jnp.transpose(fb)

    d_ref[...] = jnp.zeros((BLK, N), F32)

    def ch(cc, carry):
        base = cc * CCHUNK
        acc = d_ref[...]
        for k in range(CCHUNK):
            row = ft_ref[pl.ds(base + k, 1), :]
            col = jnp.transpose(fbt_ref[pl.ds(base + k, 1), :])
            dd = col - row
            acc = acc + dd * dd
        d_ref[...] = acc
        return carry

    lax.fori_loop(0, n_ch // CCHUNK, ch, 0)
    d_ref[...] = jnp.sqrt(jnp.maximum(d_ref[...], 0.0))

    iota = lax.broadcasted_iota(jnp.int32, (BLK, N), 1)
    rows = lax.broadcasted_iota(jnp.int32, (BLK, N), 0) + nb * BLK
    coef = _agg_coef()
    m_ref[...] = jnp.where(iota == rows, coef, F32(0.0))
    inf = F32(jnp.inf)

    def body(t, carry):
        dcur = d_ref[...]
        v = dcur
        i = iota
        for w in (512, 256, 128):
            va, vb = v[:, :w], v[:, w:]
            ia, ib = i[:, :w], i[:, w:]
            tb = (vb < va) | ((vb == va) & (ib < ia))
            v = jnp.where(tb, vb, va)
            i = jnp.where(tb, ib, ia)
        vmin = jnp.min(v, axis=1, keepdims=True)
        isel = jnp.min(jnp.where(v == vmin, i, N), axis=1, keepdims=True)
        hit = iota == isel
        d_ref[...] = jnp.where(hit, inf, dcur)
        is_s = (t >= 1) & ((t - 1) % dil == 0)

        @pl.when(is_s)
        def _():
            m_ref[...] = m_ref[...] + jnp.where(hit, coef, F32(0.0))
        return carry

    lax.fori_loop(0, n_pos, body, 0)

    msg = jnp.dot(m_ref[...], outall_ref[...], preferred_element_type=F32)
    o = msg / jnp.sqrt(F32(1.0 + EPS)) * g_ref[...] + be_ref[...]
    out_ref[0] = jnp.maximum(o, 0.0)


def _gcn_layer(feats, w, b, g, be, dil, n_ch):
    grid = (B, N // BLK)
    return pl.pallas_call(
        functools.partial(_gcn_layer_kernel, dil=dil, n_ch=n_ch),
        grid=grid,
        in_specs=[
            pl.BlockSpec((1, N, C), lambda bi, nb: (bi, 0, 0)),
            pl.BlockSpec((1, BLK, C), lambda bi, nb: (bi, nb, 0)),
            pl.BlockSpec((C, C), lambda bi, nb: (0, 0)),
            pl.BlockSpec((1, C), lambda bi, nb: (0, 0)),
            pl.BlockSpec((1, C), lambda bi, nb: (0, 0)),
            pl.BlockSpec((1, C), lambda bi, nb: (0, 0)),
        ],
        out_specs=pl.BlockSpec((1, BLK, C), lambda bi, nb: (bi, nb, 0)),
        out_shape=jax.ShapeDtypeStruct((B, N, C), F32),
        scratch_shapes=[
            pltpu.VMEM((N, C), F32),
            pltpu.VMEM((C, N), F32),
            pltpu.VMEM((C, BLK), F32),
            pltpu.VMEM((BLK, N), F32),
            pltpu.VMEM((BLK, N), F32),
        ],
    )(feats, feats, w, b, g, be)


def _head_kernel(f1_ref, f2_ref, f3_ref,
                 w1, b1, g1, be1, w2, b2, g2, be2, w3, b3, g3, be3,
                 w4, b4, g4, be4, w5, b5, g5, be5, w6, b6,
                 out_ref, pooled_ref):
    bi = pl.program_id(0)
    sqc = jnp.sqrt(F32(1.0 + EPS))
    f = (f1_ref[0] + f2_ref[0] + f3_ref[0]) / 3.0
    h = jnp.maximum((jnp.dot(f, w1[...], preferred_element_type=F32)
                     + b1[...]) / sqc * g1[...] + be1[...], 0.0)
    h = jnp.maximum((jnp.dot(h, w2[...], preferred_element_type=F32)
                     + b2[...]) / sqc * g2[...] + be2[...], 0.0)
    h = jnp.maximum((jnp.dot(h, w3[...], preferred_element_type=F32)
                     + b3[...]) / sqc * g3[...] + be3[...], 0.0)
    pooled_ref[pl.ds(bi, 1), :] = jnp.max(h, axis=0, keepdims=True)

    @pl.when(bi == B - 1)
    def _():
        pool = pooled_ref[...]
        c = jnp.maximum((jnp.dot(pool, w4[...], preferred_element_type=F32)
                         + b4[...]) / sqc * g4[...] + be4[...], 0.0)
        c = jnp.maximum((jnp.dot(c, w5[...], preferred_element_type=F32)
                         + b5[...]) / sqc * g5[...] + be5[...], 0.0)
        out_ref[...] = jnp.dot(c, w6[...], preferred_element_type=F32) + b6[...]


def _head(f1, f2, f3, p):
    r2 = lambda a: a.reshape(1, -1)
    args = [f1, f2, f3]
    specs = [pl.BlockSpec((1, N, C), lambda bi: (bi, 0, 0))] * 3
    for j in range(1, 7):
        w = p['l%d_W' % j]
        args.append(w)
        specs.append(pl.BlockSpec(w.shape, lambda bi: (0, 0)))
        bias = r2(p['l%d_b' % j])
        args.append(bias)
        specs.append(pl.BlockSpec(bias.shape, lambda bi: (0, 0)))
        if j < 6:
            for key in ('bn%d_g' % j, 'bn%d_b' % j):
                a = r2(p[key])
                args.append(a)
                specs.append(pl.BlockSpec(a.shape, lambda bi: (0, 0)))
    return pl.pallas_call(
        _head_kernel,
        grid=(B,),
        in_specs=specs,
        out_specs=pl.BlockSpec((B, 40), lambda bi: (0, 0)),
        out_shape=jax.ShapeDtypeStruct((B, 40), F32),
        scratch_shapes=[pltpu.VMEM((B, 1024), F32)],
    )(*args)


def kernel(x, params):
    p = params
    xp = jnp.pad(x, ((0, 0), (0, 0), (0, C - x.shape[-1])))
    w0 = jnp.pad(p['gcn0_W'], ((0, C - p['gcn0_W'].shape[0]), (0, 0)))
    r2 = lambda a: a.reshape(1, -1)
    f1 = _gcn_layer(xp, w0, r2(p['gcn0_b']), r2(p['gcn0_g']),
                    r2(p['gcn0_be']), 1, CCHUNK)
    f2 = _gcn_layer(f1, p['gcn1_W'], r2(p['gcn1_b']), r2(p['gcn1_g']),
                    r2(p['gcn1_be']), 2, C)
    f3 = _gcn_layer(f2, p['gcn2_W'], r2(p['gcn2_b']), r2(p['gcn2_g']),
                    r2(p['gcn2_be']), 4, C)
    return _head(f1, f2, f3, p)


# speedup vs baseline: 4.0647x; 1.1675x over previous
import functools

import jax
import jax.numpy as jnp
from jax import lax
from jax.experimental import pallas as pl
from jax.experimental.pallas import tpu as pltpu

K_NB = 16
EPS = 1e-5
B, N, C = 8, 1024, 64
BLK = 512
CCHUNK = 8
F32 = jnp.float32


def _agg_coef():
    dsib = (F32(1.0) / jnp.sqrt(F32(17.0))).astype(jnp.bfloat16).astype(F32)
    return dsib * dsib


def _gcn_layer_kernel(feats_ref, fblk_ref, w_ref, b_ref, g_ref, be_ref,
                      out_ref, outall_ref, ft_ref, fbt_ref, d_ref, m_ref,
                      *, dil, n_ch):
    nb = pl.program_id(1)
    n_pos = 1 + 15 * dil + 1

    @pl.when(nb == 0)
    def _():
        f = feats_ref[0]
        outall_ref[...] = (
            jnp.dot(f, w_ref[...], preferred_element_type=F32) + b_ref[...])
        ft_ref[...] = jnp.transpose(f)

    fb = fblk_ref[0]
    fbt_ref[...] = jnp.transpose(fb)

    d_ref[...] = jnp.zeros((BLK, N), F32)

    def ch(cc, carry):
        base = cc * CCHUNK
        acc = d_ref[...]
        for k in range(CCHUNK):
            row = ft_ref[pl.ds(base + k, 1), :]
            col = jnp.transpose(fbt_ref[pl.ds(base + k, 1), :])
            dd = col - row
            acc = acc + dd * dd
        d_ref[...] = acc
        return carry

    lax.fori_loop(0, n_ch // CCHUNK, ch, 0)
    d_ref[...] = jnp.sqrt(jnp.maximum(d_ref[...], 0.0))

    iota = lax.broadcasted_iota(jnp.int32, (BLK, N), 1)
    rows = lax.broadcasted_iota(jnp.int32, (BLK, N), 0) + nb * BLK
    coef = _agg_coef()
    m_ref[...] = jnp.where(iota == rows, coef, F32(0.0))
    inf = F32(jnp.inf)

    def body(t, carry):
        dcur = d_ref[...]
        v = dcur
        i = iota
        for w in (512, 256, 128):
            va, vb = v[:, :w], v[:, w:]
            ia, ib = i[:, :w], i[:, w:]
            tb = (vb < va) | ((vb == va) & (ib < ia))
            v = jnp.where(tb, vb, va)
            i = jnp.where(tb, ib, ia)
        vmin = jnp.min(v, axis=1, keepdims=True)
        isel = jnp.min(jnp.where(v == vmin, i, N), axis=1, keepdims=True)
        hit = iota == isel
        d_ref[...] = jnp.where(hit, inf, dcur)
        is_s = (t >= 1) & ((t - 1) % dil == 0)

        @pl.when(is_s)
        def _():
            m_ref[...] = m_ref[...] + jnp.where(hit, coef, F32(0.0))
        return carry

    lax.fori_loop(0, n_pos, body, 0)

    msg = jnp.dot(m_ref[...], outall_ref[...], preferred_element_type=F32)
    o = msg / jnp.sqrt(F32(1.0 + EPS)) * g_ref[...] + be_ref[...]
    out_ref[0] = jnp.maximum(o, 0.0)


def _gcn_layer(feats, w, b, g, be, dil, n_ch):
    grid = (B, N // BLK)
    return pl.pallas_call(
        functools.partial(_gcn_layer_kernel, dil=dil, n_ch=n_ch),
        grid=grid,
        in_specs=[
            pl.BlockSpec((1, N, C), lambda bi, nb: (bi, 0, 0)),
            pl.BlockSpec((1, BLK, C), lambda bi, nb: (bi, nb, 0)),
            pl.BlockSpec((C, C), lambda bi, nb: (0, 0)),
            pl.BlockSpec((1, C), lambda bi, nb: (0, 0)),
            pl.BlockSpec((1, C), lambda bi, nb: (0, 0)),
            pl.BlockSpec((1, C), lambda bi, nb: (0, 0)),
        ],
        out_specs=pl.BlockSpec((1, BLK, C), lambda bi, nb: (bi, nb, 0)),
        out_shape=jax.ShapeDtypeStruct((B, N, C), F32),
        scratch_shapes=[
            pltpu.VMEM((N, C), F32),
            pltpu.VMEM((C, N), F32),
            pltpu.VMEM((C, BLK), F32),
            pltpu.VMEM((BLK, N), F32),
            pltpu.VMEM((BLK, N), F32),
        ],
    )(feats, feats, w, b, g, be)


def _head_kernel(f1_ref, f2_ref, f3_ref,
                 w1, b1, g1, be1, w2, b2, g2, be2, w3, b3, g3, be3,
                 w4, b4, g4, be4, w5, b5, g5, be5, w6, b6,
                 out_ref, pooled_ref):
    bi = pl.program_id(0)
    sqc = jnp.sqrt(F32(1.0 + EPS))
    f = (f1_ref[0] + f2_ref[0] + f3_ref[0]) / 3.0
    h = jnp.maximum((jnp.dot(f, w1[...], preferred_element_type=F32)
                     + b1[...]) / sqc * g1[...] + be1[...], 0.0)
    h = jnp.maximum((jnp.dot(h, w2[...], preferred_element_type=F32)
                     + b2[...]) / sqc * g2[...] + be2[...], 0.0)
    h = jnp.maximum((jnp.dot(h, w3[...], preferred_element_type=F32)
                     + b3[...]) / sqc * g3[...] + be3[...], 0.0)
    pooled_ref[pl.ds(bi, 1), :] = jnp.max(h, axis=0, keepdims=True)

    @pl.when(bi == B - 1)
    def _():
        pool = pooled_ref[...]
        c = jnp.maximum((jnp.dot(pool, w4[...], preferred_element_type=F32)
                         + b4[...]) / sqc * g4[...] + be4[...], 0.0)
        c = jnp.maximum((jnp.dot(c, w5[...], preferred_element_type=F32)
                         + b5[...]) / sqc * g5[...] + be5[...], 0.0)
        out_ref[...] = jnp.dot(c, w6[...], preferred_element_type=F32) + b6[...]


def _head(f1, f2, f3, p):
    r2 = lambda a: a.reshape(1, -1)
    args = [f1, f2, f3]
    specs = [pl.BlockSpec((1, N, C), lambda bi: (bi, 0, 0))] * 3
    for j in range(1, 7):
        w = p['l%d_W' % j]
        args.append(w)
        specs.append(pl.BlockSpec(w.shape, lambda bi: (0, 0)))
        bias = r2(p['l%d_b' % j])
        args.append(bias)
        specs.append(pl.BlockSpec(bias.shape, lambda bi: (0, 0)))
        if j < 6:
            for key in ('bn%d_g' % j, 'bn%d_b' % j):
                a = r2(p[key])
                args.append(a)
                specs.append(pl.BlockSpec(a.shape, lambda bi: (0, 0)))
    return pl.pallas_call(
        _head_kernel,
        grid=(B,),
        in_specs=specs,
        out_specs=pl.BlockSpec((B, 40), lambda bi: (0, 0)),
        out_shape=jax.ShapeDtypeStruct((B, 40), F32),
        scratch_shapes=[pltpu.VMEM((B, 1024), F32)],
    )(*args)


def kernel(x, params):
    p = params
    xp = jnp.pad(x, ((0, 0), (0, 0), (0, C - x.shape[-1])))
    w0 = jnp.pad(p['gcn0_W'], ((0, C - p['gcn0_W'].shape[0]), (0, 0)))
    r2 = lambda a: a.reshape(1, -1)
    f1 = _gcn_layer(xp, w0, r2(p['gcn0_b']), r2(p['gcn0_g']),
                    r2(p['gcn0_be']), 1, CCHUNK)
    f2 = _gcn_layer(f1, p['gcn1_W'], r2(p['gcn1_b']), r2(p['gcn1_g']),
                    r2(p['gcn1_be']), 2, C)
    f3 = _gcn_layer(f2, p['gcn2_W'], r2(p['gcn2_b']), r2(p['gcn2_g']),
                    r2(p['gcn2_be']), 4, C)
    return _head(f1, f2, f3, p)


# speedup vs baseline: 4.2147x; 1.0369x over previous
import functools

import jax
import jax.numpy as jnp
from jax import lax
from jax.experimental import pallas as pl
from jax.experimental.pallas import tpu as pltpu

K_NB = 16
EPS = 1e-5
B, N, C = 8, 1024, 64
BLK = 1024
CCHUNK = 8
F32 = jnp.float32


def _agg_coef():
    dsib = (F32(1.0) / jnp.sqrt(F32(17.0))).astype(jnp.bfloat16).astype(F32)
    return dsib * dsib


def _gcn_layer_kernel(feats_ref, fblk_ref, w_ref, b_ref, g_ref, be_ref,
                      out_ref, outall_ref, ft_ref, fbt_ref, d_ref, m_ref,
                      *, dil, n_ch):
    nb = pl.program_id(1)
    n_pos = 1 + 15 * dil + 1

    @pl.when(nb == 0)
    def _():
        f = feats_ref[0]
        outall_ref[...] = (
            jnp.dot(f, w_ref[...], preferred_element_type=F32) + b_ref[...])
        ft_ref[...] = jnp.transpose(f)

    fb = fblk_ref[0]
    fbt_ref[...] = jnp.transpose(fb)

    d_ref[...] = jnp.zeros((BLK, N), F32)

    def ch(cc, carry):
        base = cc * CCHUNK
        acc = d_ref[...]
        for k in range(CCHUNK):
            row = ft_ref[pl.ds(base + k, 1), :]
            col = jnp.transpose(fbt_ref[pl.ds(base + k, 1), :])
            dd = col - row
            acc = acc + dd * dd
        d_ref[...] = acc
        return carry

    lax.fori_loop(0, n_ch // CCHUNK, ch, 0)
    d_ref[...] = jnp.sqrt(jnp.maximum(d_ref[...], 0.0))

    iota = lax.broadcasted_iota(jnp.int32, (BLK, N), 1)
    rows = lax.broadcasted_iota(jnp.int32, (BLK, N), 0) + nb * BLK
    coef = _agg_coef()
    m_ref[...] = jnp.where(iota == rows, coef, F32(0.0))
    inf = F32(jnp.inf)

    def body(t, carry):
        dcur = d_ref[...]
        v = dcur
        i = iota
        for w in (512, 256, 128):
            va, vb = v[:, :w], v[:, w:]
            ia, ib = i[:, :w], i[:, w:]
            tb = (vb < va) | ((vb == va) & (ib < ia))
            v = jnp.where(tb, vb, va)
            i = jnp.where(tb, ib, ia)
        vmin = jnp.min(v, axis=1, keepdims=True)
        isel = jnp.min(jnp.where(v == vmin, i, N), axis=1, keepdims=True)
        hit = iota == isel
        d_ref[...] = jnp.where(hit, inf, dcur)
        is_s = (t >= 1) & ((t - 1) % dil == 0)

        @pl.when(is_s)
        def _():
            m_ref[...] = m_ref[...] + jnp.where(hit, coef, F32(0.0))
        return carry

    lax.fori_loop(0, n_pos, body, 0)

    msg = jnp.dot(m_ref[...], outall_ref[...], preferred_element_type=F32)
    o = msg / jnp.sqrt(F32(1.0 + EPS)) * g_ref[...] + be_ref[...]
    out_ref[0] = jnp.maximum(o, 0.0)


def _gcn_layer(feats, w, b, g, be, dil, n_ch):
    grid = (B, N // BLK)
    return pl.pallas_call(
        functools.partial(_gcn_layer_kernel, dil=dil, n_ch=n_ch),
        grid=grid,
        in_specs=[
            pl.BlockSpec((1, N, C), lambda bi, nb: (bi, 0, 0)),
            pl.BlockSpec((1, BLK, C), lambda bi, nb: (bi, nb, 0)),
            pl.BlockSpec((C, C), lambda bi, nb: (0, 0)),
            pl.BlockSpec((1, C), lambda bi, nb: (0, 0)),
            pl.BlockSpec((1, C), lambda bi, nb: (0, 0)),
            pl.BlockSpec((1, C), lambda bi, nb: (0, 0)),
        ],
        out_specs=pl.BlockSpec((1, BLK, C), lambda bi, nb: (bi, nb, 0)),
        out_shape=jax.ShapeDtypeStruct((B, N, C), F32),
        scratch_shapes=[
            pltpu.VMEM((N, C), F32),
            pltpu.VMEM((C, N), F32),
            pltpu.VMEM((C, BLK), F32),
            pltpu.VMEM((BLK, N), F32),
            pltpu.VMEM((BLK, N), F32),
        ],
    )(feats, feats, w, b, g, be)


def _head_kernel(f1_ref, f2_ref, f3_ref,
                 w1, b1, g1, be1, w2, b2, g2, be2, w3, b3, g3, be3,
                 w4, b4, g4, be4, w5, b5, g5, be5, w6, b6,
                 out_ref, pooled_ref):
    bi = pl.program_id(0)
    sqc = jnp.sqrt(F32(1.0 + EPS))
    f = (f1_ref[0] + f2_ref[0] + f3_ref[0]) / 3.0
    h = jnp.maximum((jnp.dot(f, w1[...], preferred_element_type=F32)
                     + b1[...]) / sqc * g1[...] + be1[...], 0.0)
    h = jnp.maximum((jnp.dot(h, w2[...], preferred_element_type=F32)
                     + b2[...]) / sqc * g2[...] + be2[...], 0.0)
    h = jnp.maximum((jnp.dot(h, w3[...], preferred_element_type=F32)
                     + b3[...]) / sqc * g3[...] + be3[...], 0.0)
    pooled_ref[pl.ds(bi, 1), :] = jnp.max(h, axis=0, keepdims=True)

    @pl.when(bi == B - 1)
    def _():
        pool = pooled_ref[...]
        c = jnp.maximum((jnp.dot(pool, w4[...], preferred_element_type=F32)
                         + b4[...]) / sqc * g4[...] + be4[...], 0.0)
        c = jnp.maximum((jnp.dot(c, w5[...], preferred_element_type=F32)
                         + b5[...]) / sqc * g5[...] + be5[...], 0.0)
        out_ref[...] = jnp.dot(c, w6[...], preferred_element_type=F32) + b6[...]


def _head(f1, f2, f3, p):
    r2 = lambda a: a.reshape(1, -1)
    args = [f1, f2, f3]
    specs = [pl.BlockSpec((1, N, C), lambda bi: (bi, 0, 0))] * 3
    for j in range(1, 7):
        w = p['l%d_W' % j]
        args.append(w)
        specs.append(pl.BlockSpec(w.shape, lambda bi: (0, 0)))
        bias = r2(p['l%d_b' % j])
        args.append(bias)
        specs.append(pl.BlockSpec(bias.shape, lambda bi: (0, 0)))
        if j < 6:
            for key in ('bn%d_g' % j, 'bn%d_b' % j):
                a = r2(p[key])
                args.append(a)
                specs.append(pl.BlockSpec(a.shape, lambda bi: (0, 0)))
    return pl.pallas_call(
        _head_kernel,
        grid=(B,),
        in_specs=specs,
        out_specs=pl.BlockSpec((B, 40), lambda bi: (0, 0)),
        out_shape=jax.ShapeDtypeStruct((B, 40), F32),
        scratch_shapes=[pltpu.VMEM((B, 1024), F32)],
    )(*args)


def kernel(x, params):
    p = params
    xp = jnp.pad(x, ((0, 0), (0, 0), (0, C - x.shape[-1])))
    w0 = jnp.pad(p['gcn0_W'], ((0, C - p['gcn0_W'].shape[0]), (0, 0)))
    r2 = lambda a: a.reshape(1, -1)
    f1 = _gcn_layer(xp, w0, r2(p['gcn0_b']), r2(p['gcn0_g']),
                    r2(p['gcn0_be']), 1, CCHUNK)
    f2 = _gcn_layer(f1, p['gcn1_W'], r2(p['gcn1_b']), r2(p['gcn1_g']),
                    r2(p['gcn1_be']), 2, C)
    f3 = _gcn_layer(f2, p['gcn2_W'], r2(p['gcn2_b']), r2(p['gcn2_g']),
                    r2(p['gcn2_be']), 4, C)
    return _head(f1, f2, f3, p)
